# Initial kernel scaffold; baseline (speedup 1.0000x reference)
import functools

import jax
import jax.numpy as jnp
from jax import lax
from jax.experimental import pallas as pl
from jax.experimental.pallas import tpu as pltpu
from jax.experimental.pallas import tpu_sc as plsc

N = 10000
NFEAT = 128
HID = 16
H1 = 8
NCLASS = 40

E_RAW = 320000
E_REAL = E_RAW + N
EP = 331776
NW = 32
R = 4
ECR = EP // (NW * R)
BB = 32
NB = ECR // BB
ZR = N // 16

BN = 1000


def _lrelu(v):
    return jnp.where(v > 0, v, 0.2 * v)


def _pre1_body(x_ref, w_ref, as_ref, ad_ref, xl_ref, asT_ref, adT_ref, gm_ref):
    i = pl.program_id(0)
    xl = jnp.dot(x_ref[...], w_ref[...], preferred_element_type=jnp.float32)
    xl_ref[...] = xl
    a_s = jnp.dot(xl, as_ref[...], preferred_element_type=jnp.float32)
    a_d = jnp.dot(xl, ad_ref[...], preferred_element_type=jnp.float32)
    asT_ref[...] = a_s.T
    adT_ref[...] = a_d.T
    bm = jnp.max(a_s, axis=0)
    cur = jnp.broadcast_to(bm[:, None], (H1, 128))

    @pl.when(i == 0)
    def _():
        gm_ref[...] = cur

    @pl.when(i > 0)
    def _():
        gm_ref[...] = jnp.maximum(gm_ref[...], cur)


_pre1 = pl.pallas_call(
    _pre1_body,
    grid=(N // BN,),
    in_specs=[
        pl.BlockSpec((BN, NFEAT), lambda i: (i, 0)),
        pl.BlockSpec((NFEAT, NFEAT), lambda i: (0, 0)),
        pl.BlockSpec((NFEAT, H1), lambda i: (0, 0)),
        pl.BlockSpec((NFEAT, H1), lambda i: (0, 0)),
    ],
    out_specs=[
        pl.BlockSpec((BN, NFEAT), lambda i: (i, 0)),
        pl.BlockSpec((H1, BN), lambda i: (0, i)),
        pl.BlockSpec((H1, BN), lambda i: (0, i)),
        pl.BlockSpec((H1, 128), lambda i: (0, 0)),
    ],
    out_shape=[
        jax.ShapeDtypeStruct((N, NFEAT), jnp.float32),
        jax.ShapeDtypeStruct((H1, N), jnp.float32),
        jax.ShapeDtypeStruct((H1, N), jnp.float32),
        jax.ShapeDtypeStruct((H1, 128), jnp.float32),
    ],
)


def _make_edge_kernel(H, CROW, UROW):
    mesh = plsc.VectorSubcoreMesh(core_axis_name="c", subcore_axis_name="s")

    def assemble(e, _, off, p_v, rows_v, upd_v):
        pcol = jnp.full((16,), off + e, jnp.int32)
        if H == H1:
            for h in range(H):
                pb = plsc.load_gather(p_v, [jnp.full((16,), h, jnp.int32), pcol])
                upd_v[e, pl.ds(h * 16, 16)] = rows_v[e, pl.ds(h * 16, 16)] * pb
            hidx = lax.iota(jnp.int32, 16)
            pall = plsc.load_gather(p_v, [jnp.minimum(hidx, H - 1), pcol])
            upd_v[e, pl.ds(H * 16, 16)] = jnp.where(
                hidx < H, pall, jnp.zeros((16,), jnp.float32))
        else:
            pb = plsc.load_gather(p_v, [jnp.zeros((16,), jnp.int32), pcol])
            onehot = jnp.where(lax.iota(jnp.int32, 16) == 8,
                               jnp.ones((16,), jnp.float32),
                               jnp.zeros((16,), jnp.float32))
            upd_v[e, pl.ds(0, 16)] = rows_v[e, pl.ds(0, 16)] * pb
            upd_v[e, pl.ds(16, 16)] = rows_v[e, pl.ds(16, 16)] * pb
            upd_v[e, pl.ds(32, 16)] = (rows_v[e, pl.ds(32, 16)] + onehot) * pb
        return 0

    def body(srcp, dstp, xl, asT, adT, gmaxT, zrow, parts,
             acc, src_v, dst_v, p_v, as_v, ad_v, gm_v, sidx, didx,
             rows_v, upd_v, sem):
        c = lax.axis_index("c")
        s = lax.axis_index("s")
        w = s * 2 + c
        pltpu.sync_copy(zrow, acc.at[pl.ds(s * ZR, ZR)])
        pltpu.sync_copy(gmaxT, gm_v)
        plsc.subcore_barrier()

        def round_body(r, _):
            base = (w * R + r) * ECR
            pltpu.sync_copy(srcp.at[pl.ds(base, ECR)], src_v)
            pltpu.sync_copy(dstp.at[pl.ds(base, ECR)], dst_v)
            for h in range(H):
                pltpu.sync_copy(asT.at[h], as_v)
                pltpu.sync_copy(adT.at[h], ad_v)
                gmv = gm_v[h]

                def p_body(i, _, h=h, gmv=gmv, base=base):
                    si = src_v[pl.ds(i * 16, 16)]
                    di = dst_v[pl.ds(i * 16, 16)]
                    sv = plsc.load_gather(as_v, [si])
                    dv = plsc.load_gather(ad_v, [di])
                    al = _lrelu(sv + dv)
                    uu = _lrelu(gmv + dv)
                    p = jnp.exp(al - uu)
                    eid = base + i * 16 + lax.iota(jnp.int32, 16)
                    p = jnp.where(eid < E_REAL, p,
                                  jnp.zeros((16,), jnp.float32))
                    p_v[h, pl.ds(i * 16, 16)] = p
                    return 0

                lax.fori_loop(0, ECR // 16, p_body, 0)

            def b_body(b, _):
                off = b * BB
                pltpu.sync_copy(src_v.at[pl.ds(off, BB)], sidx)
                pltpu.sync_copy(dst_v.at[pl.ds(off, BB)], didx)
                pltpu.async_copy(xl.at[sidx], rows_v, sem).wait()
                lax.fori_loop(0, BB,
                              functools.partial(assemble, off=off, p_v=p_v,
                                                rows_v=rows_v, upd_v=upd_v),
                              0)
                pltpu.sync_copy(upd_v, acc.at[didx], add=True)
                return 0

            lax.fori_loop(0, NB, b_body, 0)
            return 0

        lax.fori_loop(0, R, round_body, 0)
        plsc.subcore_barrier()
        pltpu.sync_copy(acc.at[pl.ds(s * ZR, ZR)],
                        parts.at[c].at[pl.ds(s * ZR, ZR)])

    return pl.kernel(
        body,
        out_type=jax.ShapeDtypeStruct((2, N, UROW), jnp.float32),
        mesh=mesh,
        scratch_types=[
            pltpu.VMEM_SHARED((N, UROW), jnp.float32),
            pltpu.VMEM((ECR,), jnp.int32),
            pltpu.VMEM((ECR,), jnp.int32),
            pltpu.VMEM((H, ECR), jnp.float32),
            pltpu.VMEM((N,), jnp.float32),
            pltpu.VMEM((N,), jnp.float32),
            pltpu.VMEM((H, 16), jnp.float32),
            pltpu.VMEM((BB,), jnp.int32),
            pltpu.VMEM((BB,), jnp.int32),
            pltpu.VMEM((BB, CROW), jnp.float32),
            pltpu.VMEM((BB, UROW), jnp.float32),
            pltpu.SemaphoreType.DMA,
        ],
    )


_edge1 = _make_edge_kernel(H1, 128, 144)
_edge2 = _make_edge_kernel(1, 48, 48)


def _mid_body(p0_ref, p1_ref, b1_ref, sexp_ref, w2_ref, as2_ref, ad2_ref,
              xl2_ref, s2_ref, d2_ref, gm2_ref):
    i = pl.program_id(0)
    raw = p0_ref[:, pl.ds(0, 128)] + p1_ref[:, pl.ds(0, 128)]
    den8 = p0_ref[:, pl.ds(128, 16)] + p1_ref[:, pl.ds(128, 16)]
    den128 = jnp.dot(den8, sexp_ref[...], preferred_element_type=jnp.float32)
    h = raw / (den128 + 1e-16) + b1_ref[...]
    h = jnp.where(h > 0, h, jnp.expm1(h))
    xl2 = jnp.dot(h, w2_ref[...], preferred_element_type=jnp.float32)
    xl2_ref[...] = xl2
    s2 = jnp.sum(xl2 * as2_ref[...], axis=1, keepdims=True)
    d2 = jnp.sum(xl2 * ad2_ref[...], axis=1, keepdims=True)
    s2_ref[...] = s2
    d2_ref[...] = d2
    cur = jnp.full((8, 128), jnp.max(s2), jnp.float32)

    @pl.when(i == 0)
    def _():
        gm2_ref[...] = cur

    @pl.when(i > 0)
    def _():
        gm2_ref[...] = jnp.maximum(gm2_ref[...], cur)


_mid = pl.pallas_call(
    _mid_body,
    grid=(N // BN,),
    in_specs=[
        pl.BlockSpec((BN, 144), lambda i: (i, 0)),
        pl.BlockSpec((BN, 144), lambda i: (i, 0)),
        pl.BlockSpec((1, 128), lambda i: (0, 0)),
        pl.BlockSpec((16, 128), lambda i: (0, 0)),
        pl.BlockSpec((NFEAT, 48), lambda i: (0, 0)),
        pl.BlockSpec((1, 48), lambda i: (0, 0)),
        pl.BlockSpec((1, 48), lambda i: (0, 0)),
    ],
    out_specs=[
        pl.BlockSpec((BN, 48), lambda i: (i, 0)),
        pl.BlockSpec((BN, 1), lambda i: (i, 0)),
        pl.BlockSpec((BN, 1), lambda i: (i, 0)),
        pl.BlockSpec((8, 128), lambda i: (0, 0)),
    ],
    out_shape=[
        jax.ShapeDtypeStruct((N, 48), jnp.float32),
        jax.ShapeDtypeStruct((N, 1), jnp.float32),
        jax.ShapeDtypeStruct((N, 1), jnp.float32),
        jax.ShapeDtypeStruct((8, 128), jnp.float32),
    ],
)


def _fin_body(q0_ref, q1_ref, b2_ref, out_ref):
    raw = q0_ref[...] + q1_ref[...]
    den = raw[:, pl.ds(40, 1)]
    out_ref[...] = raw / (den + 1e-16) + b2_ref[...]


_fin = pl.pallas_call(
    _fin_body,
    grid=(N // BN,),
    in_specs=[
        pl.BlockSpec((BN, 48), lambda i: (i, 0)),
        pl.BlockSpec((BN, 48), lambda i: (i, 0)),
        pl.BlockSpec((1, 48), lambda i: (0, 0)),
    ],
    out_specs=pl.BlockSpec((BN, 48), lambda i: (i, 0)),
    out_shape=jax.ShapeDtypeStruct((N, 48), jnp.float32),
)


def kernel(x, edge_index, W1, att_src1, att_dst1, b1, W2, att_src2, att_dst2,
           b2):
    f32 = jnp.float32
    loop = jnp.arange(N, dtype=edge_index.dtype)
    pad = jnp.zeros((EP - E_REAL,), edge_index.dtype)
    srcp = jnp.concatenate([edge_index[0], loop, pad])
    dstp = jnp.concatenate([edge_index[1], loop, pad])

    rows = jnp.arange(NFEAT)
    As1 = jnp.zeros((NFEAT, H1), f32).at[rows, rows // HID].set(
        att_src1.reshape(-1))
    Ad1 = jnp.zeros((NFEAT, H1), f32).at[rows, rows // HID].set(
        att_dst1.reshape(-1))
    Sexp = jnp.zeros((16, 128), f32).at[rows // HID, rows].set(1.0)

    xl1, asT1, adT1, gm1 = _pre1(x, W1, As1, Ad1)
    gmT1 = gm1[:, :HID]

    parts1 = _edge1(srcp, dstp, xl1, asT1, adT1, gmT1,
                    jnp.zeros((ZR, 144), f32))

    W2p = jnp.zeros((NFEAT, 48), f32).at[:, :NCLASS].set(W2)
    as2row = jnp.zeros((1, 48), f32).at[:, :NCLASS].set(att_src2)
    ad2row = jnp.zeros((1, 48), f32).at[:, :NCLASS].set(att_dst2)
    xl2, s2, d2, gm2 = _mid(parts1[0], parts1[1], b1[None, :], Sexp, W2p,
                            as2row, ad2row)
    gmT2 = gm2[:1, :16]

    parts2 = _edge2(srcp, dstp, xl2, s2.reshape(1, N), d2.reshape(1, N), gmT2,
                    jnp.zeros((ZR, 48), f32))

    b2p = jnp.zeros((1, 48), f32).at[:, :NCLASS].set(b2)
    out48 = _fin(parts2[0], parts2[1], b2p)
    return out48[:, :NCLASS]


# speedup vs baseline: 32.9557x; 32.9557x over previous
import functools

import jax
import jax.numpy as jnp
from jax import lax
from jax.experimental import pallas as pl
from jax.experimental.pallas import tpu as pltpu
from jax.experimental.pallas import tpu_sc as plsc

N = 10000
NFEAT = 128
HID = 16
H1 = 8
NCLASS = 40

E_RAW = 320000
E_REAL = E_RAW + N
EP = 331776
NW = 32
R = 4
ECR = EP // (NW * R)
BB = 32
NB = ECR // BB
NACC = 10240
ZR = NACC // 16

BN = 1000
NPAD = 10240
BN1 = 1024


def _lrelu(v):
    return jnp.where(v > 0, v, 0.2 * v)


def _pre1_body(x_ref, w_ref, as_ref, ad_ref, xl_ref, asT_ref, adT_ref, gm_ref):
    i = pl.program_id(0)
    xl = jnp.dot(x_ref[...], w_ref[...], preferred_element_type=jnp.float32)
    xl_ref[...] = xl
    a_s = jnp.dot(xl, as_ref[...], preferred_element_type=jnp.float32)
    a_d = jnp.dot(xl, ad_ref[...], preferred_element_type=jnp.float32)
    asT_ref[...] = a_s.T
    adT_ref[...] = a_d.T
    bm = jnp.max(a_s, axis=0)
    cur = jnp.broadcast_to(bm[:, None], (H1, 128))

    @pl.when(i == 0)
    def _():
        gm_ref[...] = cur

    @pl.when(i > 0)
    def _():
        gm_ref[...] = jnp.maximum(gm_ref[...], cur)


_pre1 = pl.pallas_call(
    _pre1_body,
    grid=(NPAD // BN1,),
    in_specs=[
        pl.BlockSpec((BN1, NFEAT), lambda i: (i, 0)),
        pl.BlockSpec((NFEAT, NFEAT), lambda i: (0, 0)),
        pl.BlockSpec((NFEAT, H1), lambda i: (0, 0)),
        pl.BlockSpec((NFEAT, H1), lambda i: (0, 0)),
    ],
    out_specs=[
        pl.BlockSpec((BN1, NFEAT), lambda i: (i, 0)),
        pl.BlockSpec((H1, BN1), lambda i: (0, i)),
        pl.BlockSpec((H1, BN1), lambda i: (0, i)),
        pl.BlockSpec((H1, 128), lambda i: (0, 0)),
    ],
    out_shape=[
        jax.ShapeDtypeStruct((NPAD, NFEAT), jnp.float32),
        jax.ShapeDtypeStruct((H1, NPAD), jnp.float32),
        jax.ShapeDtypeStruct((H1, NPAD), jnp.float32),
        jax.ShapeDtypeStruct((H1, 128), jnp.float32),
    ],
)


WEC = EP // NW
RNB = WEC // BB

_MESH = plsc.VectorSubcoreMesh(core_axis_name="c", subcore_axis_name="s",
                               num_cores=2, num_subcores=16)
_SC_PARAMS = pltpu.CompilerParams(use_tc_tiling_on_sc=False,
                                  needs_layout_passes=False)


def _make_p_kernel(H, TN):
    def body(srcp, dstp, asT, adT, gmaxT, ph,
             src_v, dst_v, as_v, ad_v, gm_v, p_r):
        c = lax.axis_index("c")
        s = lax.axis_index("s")
        w = s * 2 + c
        pltpu.sync_copy(gmaxT, gm_v)
        pltpu.sync_copy(srcp.at[pl.ds(w * RNB, RNB)], src_v)
        pltpu.sync_copy(dstp.at[pl.ds(w * RNB, RNB)], dst_v)
        base = w * WEC
        for h in range(H):
            pltpu.sync_copy(asT.at[h], as_v)
            pltpu.sync_copy(adT.at[h], ad_v)
            gmv = gm_v[h]

            def p_body(b, _, gmv=gmv):
                for j in range(BB // 16):
                    si = src_v[b, pl.ds(j * 16, 16)]
                    di = dst_v[b, pl.ds(j * 16, 16)]
                    sv = plsc.load_gather(as_v, [si])
                    dv = plsc.load_gather(ad_v, [di])
                    al = _lrelu(sv + dv)
                    uu = _lrelu(gmv + dv)
                    p = jnp.exp(al - uu)
                    eid = base + b * BB + j * 16 + lax.iota(jnp.int32, 16)
                    p = jnp.where(eid < E_REAL, p,
                                  jnp.zeros((16,), jnp.float32))
                    p_r[pl.ds(b * BB + j * 16, 16)] = p
                return 0

            lax.fori_loop(0, RNB, p_body, 0)
            pltpu.sync_copy(p_r, ph.at[h, pl.ds(base, WEC)])

    return pl.kernel(
        body,
        out_type=jax.ShapeDtypeStruct((H, EP), jnp.float32),
        mesh=_MESH,
        compiler_params=_SC_PARAMS,
        scratch_types=[
            pltpu.VMEM((RNB, BB), jnp.int32),
            pltpu.VMEM((RNB, BB), jnp.int32),
            pltpu.VMEM((TN,), jnp.float32),
            pltpu.VMEM((TN,), jnp.float32),
            pltpu.VMEM((H, 16), jnp.float32),
            pltpu.VMEM((WEC,), jnp.float32),
        ],
    )


def _make_s_kernel(H, CROW, UROW):
    def assemble(e, _, off, p_r, rows_v, upd_v):
        pcol = jnp.full((16,), off + e, jnp.int32)
        if H == H1:
            for h in range(H):
                pb = plsc.load_gather(p_r, [jnp.full((16,), h, jnp.int32), pcol])
                upd_v[e, pl.ds(h * 16, 16)] = rows_v[e, pl.ds(h * 16, 16)] * pb
            hidx = lax.iota(jnp.int32, 16)
            pall = plsc.load_gather(p_r, [jnp.minimum(hidx, H - 1), pcol])
            upd_v[e, pl.ds(H * 16, 16)] = jnp.where(
                hidx < H, pall, jnp.zeros((16,), jnp.float32))
        else:
            pb = plsc.load_gather(p_r, [jnp.zeros((16,), jnp.int32), pcol])
            onehot = jnp.where(lax.iota(jnp.int32, 16) == 8,
                               jnp.ones((16,), jnp.float32),
                               jnp.zeros((16,), jnp.float32))
            upd_v[e, pl.ds(0, 16)] = rows_v[e, pl.ds(0, 16)] * pb
            upd_v[e, pl.ds(16, 16)] = rows_v[e, pl.ds(16, 16)] * pb
            upd_v[e, pl.ds(32, 16)] = (rows_v[e, pl.ds(32, 16)] + onehot) * pb
        return 0

    def body(srcp, dstp, xl, ph, zrow, parts,
             acc, src_v, dst_v, p_r, rows_v, upd_v, sem):
        c = lax.axis_index("c")
        s = lax.axis_index("s")
        w = s * 2 + c
        pltpu.sync_copy(zrow, acc.at[pl.ds(s * ZR, ZR)])
        plsc.subcore_barrier()

        def round_body(r, _):
            rowbase = (w * R + r) * NB
            base = rowbase * BB
            pltpu.sync_copy(srcp.at[pl.ds(rowbase, NB)], src_v)
            pltpu.sync_copy(dstp.at[pl.ds(rowbase, NB)], dst_v)
            pltpu.sync_copy(ph.at[:, pl.ds(base, ECR)], p_r)

            def b_body(b, _):
                pltpu.async_copy(xl.at[src_v.at[b]], rows_v, sem).wait()
                lax.fori_loop(0, BB,
                              functools.partial(assemble, off=b * BB, p_r=p_r,
                                                rows_v=rows_v, upd_v=upd_v),
                              0)
                pltpu.sync_copy(upd_v, acc.at[dst_v.at[b]], add=True)
                return 0

            lax.fori_loop(0, NB, b_body, 0)
            return 0

        lax.fori_loop(0, R, round_body, 0)
        plsc.subcore_barrier()
        pltpu.sync_copy(acc.at[pl.ds(s * ZR, ZR)],
                        parts.at[c, pl.ds(s * ZR, ZR)])

    return pl.kernel(
        body,
        out_type=jax.ShapeDtypeStruct((2, NACC, UROW), jnp.float32),
        mesh=_MESH,
        compiler_params=_SC_PARAMS,
        scratch_types=[
            pltpu.VMEM_SHARED((NACC, UROW), jnp.float32),
            pltpu.VMEM((NB, BB), jnp.int32),
            pltpu.VMEM((NB, BB), jnp.int32),
            pltpu.VMEM((H, ECR), jnp.float32),
            pltpu.VMEM((BB, CROW), jnp.float32),
            pltpu.VMEM((BB, UROW), jnp.float32),
            pltpu.SemaphoreType.DMA,
        ],
    )


_p1 = _make_p_kernel(H1, NPAD)
_p2 = _make_p_kernel(1, N)
_s1 = _make_s_kernel(H1, 128, 144)
_s2 = _make_s_kernel(1, 48, 48)


def _mid_body(p0_ref, p1_ref, b1_ref, sexp_ref, w2_ref, as2_ref, ad2_ref,
              xl2_ref, s2_ref, d2_ref, gm2_ref):
    i = pl.program_id(0)
    raw = p0_ref[:, pl.ds(0, 128)] + p1_ref[:, pl.ds(0, 128)]
    den8 = p0_ref[:, pl.ds(128, 16)] + p1_ref[:, pl.ds(128, 16)]
    den128 = jnp.dot(den8, sexp_ref[...], preferred_element_type=jnp.float32)
    h = raw / (den128 + 1e-16) + b1_ref[...]
    h = jnp.where(h > 0, h, jnp.exp(h) - 1.0)
    xl2 = jnp.dot(h, w2_ref[...], preferred_element_type=jnp.float32)
    xl2_ref[...] = xl2
    s2 = jnp.sum(xl2 * as2_ref[...], axis=1, keepdims=True)
    d2 = jnp.sum(xl2 * ad2_ref[...], axis=1, keepdims=True)
    s2_ref[...] = s2
    d2_ref[...] = d2
    cur = jnp.full((8, 128), jnp.max(s2), jnp.float32)

    @pl.when(i == 0)
    def _():
        gm2_ref[...] = cur

    @pl.when(i > 0)
    def _():
        gm2_ref[...] = jnp.maximum(gm2_ref[...], cur)


_mid = pl.pallas_call(
    _mid_body,
    grid=(N // BN,),
    in_specs=[
        pl.BlockSpec((BN, 144), lambda i: (i, 0)),
        pl.BlockSpec((BN, 144), lambda i: (i, 0)),
        pl.BlockSpec((1, 128), lambda i: (0, 0)),
        pl.BlockSpec((16, 128), lambda i: (0, 0)),
        pl.BlockSpec((NFEAT, 48), lambda i: (0, 0)),
        pl.BlockSpec((1, 48), lambda i: (0, 0)),
        pl.BlockSpec((1, 48), lambda i: (0, 0)),
    ],
    out_specs=[
        pl.BlockSpec((BN, 48), lambda i: (i, 0)),
        pl.BlockSpec((BN, 1), lambda i: (i, 0)),
        pl.BlockSpec((BN, 1), lambda i: (i, 0)),
        pl.BlockSpec((8, 128), lambda i: (0, 0)),
    ],
    out_shape=[
        jax.ShapeDtypeStruct((N, 48), jnp.float32),
        jax.ShapeDtypeStruct((N, 1), jnp.float32),
        jax.ShapeDtypeStruct((N, 1), jnp.float32),
        jax.ShapeDtypeStruct((8, 128), jnp.float32),
    ],
)


def _fin_body(q0_ref, q1_ref, b2_ref, out_ref):
    raw = q0_ref[...] + q1_ref[...]
    den = raw[:, 40:41]
    out_ref[...] = raw / (den + 1e-16) + b2_ref[...]


_fin = pl.pallas_call(
    _fin_body,
    grid=(N // BN,),
    in_specs=[
        pl.BlockSpec((BN, 48), lambda i: (i, 0)),
        pl.BlockSpec((BN, 48), lambda i: (i, 0)),
        pl.BlockSpec((1, 48), lambda i: (0, 0)),
    ],
    out_specs=pl.BlockSpec((BN, 48), lambda i: (i, 0)),
    out_shape=jax.ShapeDtypeStruct((N, 48), jnp.float32),
)


def kernel(x, edge_index, W1, att_src1, att_dst1, b1, W2, att_src2, att_dst2,
           b2):
    f32 = jnp.float32
    loop = jnp.arange(N, dtype=edge_index.dtype)
    pad = jnp.zeros((EP - E_REAL,), edge_index.dtype)
    srcp = jnp.concatenate([edge_index[0], loop, pad]).reshape(EP // BB, BB)
    dstp = jnp.concatenate([edge_index[1], loop, pad]).reshape(EP // BB, BB)

    rows = jnp.arange(NFEAT)
    As1 = jnp.zeros((NFEAT, H1), f32).at[rows, rows // HID].set(
        att_src1.reshape(-1))
    Ad1 = jnp.zeros((NFEAT, H1), f32).at[rows, rows // HID].set(
        att_dst1.reshape(-1))
    Sexp = jnp.zeros((16, 128), f32).at[rows // HID, rows].set(1.0)

    xpad = jnp.concatenate([x, jnp.zeros((NPAD - N, NFEAT), f32)])
    xl1, asT1, adT1, gm1 = _pre1(xpad, W1, As1, Ad1)
    gmT1 = gm1[:, :HID]

    ph1 = _p1(srcp, dstp, asT1, adT1, gmT1)
    parts1 = _s1(srcp, dstp, xl1, ph1, jnp.zeros((ZR, 144), f32))

    W2p = jnp.zeros((NFEAT, 48), f32).at[:, :NCLASS].set(W2)
    as2row = jnp.zeros((1, 48), f32).at[:, :NCLASS].set(att_src2)
    ad2row = jnp.zeros((1, 48), f32).at[:, :NCLASS].set(att_dst2)
    xl2, s2, d2, gm2 = _mid(parts1[0, :N], parts1[1, :N], b1[None, :], Sexp,
                            W2p, as2row, ad2row)
    gmT2 = gm2[:1, :16]

    ph2 = _p2(srcp, dstp, s2.reshape(1, N), d2.reshape(1, N), gmT2)
    parts2 = _s2(srcp, dstp, xl2, ph2, jnp.zeros((ZR, 48), f32))

    b2p = jnp.zeros((1, 48), f32).at[:, :NCLASS].set(b2)
    out48 = _fin(parts2[0, :N], parts2[1, :N], b2p)
    return out48[:, :NCLASS]


# speedup vs baseline: 47.9793x; 1.4559x over previous
import functools

import jax
import jax.numpy as jnp
from jax import lax
from jax.experimental import pallas as pl
from jax.experimental.pallas import tpu as pltpu
from jax.experimental.pallas import tpu_sc as plsc

N = 10000
NFEAT = 128
HID = 16
H1 = 8
NCLASS = 40

E_RAW = 320000
E_REAL = E_RAW + N
EP = 331776
NW = 32
R = 6
ECR = EP // (NW * R)
BB = 32
NB = ECR // BB
NACC = 10240
ZR = NACC // 16

BN = 1000
NPAD = 10240
BN1 = 1024


def _lrelu(v):
    return jnp.where(v > 0, v, 0.2 * v)


def _pre1_body(x_ref, w_ref, as_ref, ad_ref, xl_ref, asT_ref, adT_ref, gm_ref):
    i = pl.program_id(0)
    xl = jnp.dot(x_ref[...], w_ref[...], preferred_element_type=jnp.float32)
    xl_ref[...] = xl
    a_s = jnp.dot(xl, as_ref[...], preferred_element_type=jnp.float32)
    a_d = jnp.dot(xl, ad_ref[...], preferred_element_type=jnp.float32)
    asT_ref[...] = a_s.T
    adT_ref[...] = a_d.T
    bm = jnp.max(a_s, axis=0)
    cur = jnp.broadcast_to(bm[:, None], (H1, 128))

    @pl.when(i == 0)
    def _():
        gm_ref[...] = cur

    @pl.when(i > 0)
    def _():
        gm_ref[...] = jnp.maximum(gm_ref[...], cur)


_pre1 = pl.pallas_call(
    _pre1_body,
    grid=(NPAD // BN1,),
    in_specs=[
        pl.BlockSpec((BN1, NFEAT), lambda i: (i, 0)),
        pl.BlockSpec((NFEAT, NFEAT), lambda i: (0, 0)),
        pl.BlockSpec((NFEAT, H1), lambda i: (0, 0)),
        pl.BlockSpec((NFEAT, H1), lambda i: (0, 0)),
    ],
    out_specs=[
        pl.BlockSpec((BN1, NFEAT), lambda i: (i, 0)),
        pl.BlockSpec((H1, BN1), lambda i: (0, i)),
        pl.BlockSpec((H1, BN1), lambda i: (0, i)),
        pl.BlockSpec((H1, 128), lambda i: (0, 0)),
    ],
    out_shape=[
        jax.ShapeDtypeStruct((NPAD, NFEAT), jnp.float32),
        jax.ShapeDtypeStruct((H1, NPAD), jnp.float32),
        jax.ShapeDtypeStruct((H1, NPAD), jnp.float32),
        jax.ShapeDtypeStruct((H1, 128), jnp.float32),
    ],
)


WEC = EP // NW
RNB = WEC // BB

_MESH = plsc.VectorSubcoreMesh(core_axis_name="c", subcore_axis_name="s",
                               num_cores=2, num_subcores=16)
_SC_PARAMS = pltpu.CompilerParams(use_tc_tiling_on_sc=False,
                                  needs_layout_passes=False)


def _make_p_kernel(H, TN):
    def body(srcp, dstp, asT, adT, gmaxT, ph,
             src_v, dst_v, as_v, ad_v, gm_v, p_r):
        c = lax.axis_index("c")
        s = lax.axis_index("s")
        w = s * 2 + c
        pltpu.sync_copy(gmaxT, gm_v)
        pltpu.sync_copy(srcp.at[pl.ds(w * RNB, RNB)], src_v)
        pltpu.sync_copy(dstp.at[pl.ds(w * RNB, RNB)], dst_v)
        base = w * WEC
        for h in range(H):
            pltpu.sync_copy(asT.at[h], as_v)
            pltpu.sync_copy(adT.at[h], ad_v)
            gmv = gm_v[h]

            def p_body(b, _, gmv=gmv):
                for j in range(BB // 16):
                    si = src_v[b, pl.ds(j * 16, 16)]
                    di = dst_v[b, pl.ds(j * 16, 16)]
                    sv = plsc.load_gather(as_v, [si])
                    dv = plsc.load_gather(ad_v, [di])
                    al = _lrelu(sv + dv)
                    uu = _lrelu(gmv + dv)
                    p = jnp.exp(al - uu)
                    eid = base + b * BB + j * 16 + lax.iota(jnp.int32, 16)
                    p = jnp.where(eid < E_REAL, p,
                                  jnp.zeros((16,), jnp.float32))
                    p_r[pl.ds(b * BB + j * 16, 16)] = p
                return 0

            lax.fori_loop(0, RNB, p_body, 0)
            pltpu.sync_copy(p_r, ph.at[h, pl.ds(base, WEC)])

    return pl.kernel(
        body,
        out_type=jax.ShapeDtypeStruct((H, EP), jnp.float32),
        mesh=_MESH,
        compiler_params=_SC_PARAMS,
        scratch_types=[
            pltpu.VMEM((RNB, BB), jnp.int32),
            pltpu.VMEM((RNB, BB), jnp.int32),
            pltpu.VMEM((TN,), jnp.float32),
            pltpu.VMEM((TN,), jnp.float32),
            pltpu.VMEM((H, 16), jnp.float32),
            pltpu.VMEM((WEC,), jnp.float32),
        ],
    )


def _make_s_kernel(H, CROW, UROW):
    def assemble(e, _, off, p_r, rows_v, upd_v):
        pcol = jnp.full((16,), off + e, jnp.int32)
        if H == H1:
            for h in range(H):
                pb = plsc.load_gather(p_r, [jnp.full((16,), h, jnp.int32), pcol])
                upd_v[e, pl.ds(h * 16, 16)] = rows_v[e, pl.ds(h * 16, 16)] * pb
            hidx = lax.iota(jnp.int32, 16)
            pall = plsc.load_gather(p_r, [jnp.minimum(hidx, H - 1), pcol])
            upd_v[e, pl.ds(H * 16, 16)] = jnp.where(
                hidx < H, pall, jnp.zeros((16,), jnp.float32))
        else:
            pb = plsc.load_gather(p_r, [jnp.zeros((16,), jnp.int32), pcol])
            onehot = jnp.where(lax.iota(jnp.int32, 16) == 8,
                               jnp.ones((16,), jnp.float32),
                               jnp.zeros((16,), jnp.float32))
            upd_v[e, pl.ds(0, 16)] = rows_v[e, pl.ds(0, 16)] * pb
            upd_v[e, pl.ds(16, 16)] = rows_v[e, pl.ds(16, 16)] * pb
            upd_v[e, pl.ds(32, 16)] = (rows_v[e, pl.ds(32, 16)] + onehot) * pb
        return 0

    def body(srcp, dstp, xl, ph, zrow, parts,
             acc, src_v, dst_v, p_r, rows0, rows1, upd0, upd1,
             gsem0, gsem1, ssem0, ssem1):
        c = lax.axis_index("c")
        s = lax.axis_index("s")
        w = s * 2 + c
        rows = (rows0, rows1)
        upd = (upd0, upd1)
        gsem = (gsem0, gsem1)
        ssem = (ssem0, ssem1)
        pltpu.sync_copy(zrow, acc.at[pl.ds(s * ZR, ZR)])
        plsc.subcore_barrier()

        def round_body(r, _):
            rowbase = (w * R + r) * NB
            base = rowbase * BB
            pltpu.sync_copy(srcp.at[pl.ds(rowbase, NB)], src_v)
            pltpu.sync_copy(dstp.at[pl.ds(rowbase, NB)], dst_v)
            pltpu.sync_copy(ph.at[:, pl.ds(base, ECR)], p_r)

            def zero_upd(e, _):
                zv = jnp.zeros((16,), jnp.float32)
                for k in range(UROW // 16):
                    upd[0][e, pl.ds(k * 16, 16)] = zv
                    upd[1][e, pl.ds(k * 16, 16)] = zv
                return 0

            lax.fori_loop(0, BB, zero_upd, 0)
            pltpu.async_copy(upd[0], acc.at[dst_v.at[0]], ssem[0], add=True)
            pltpu.async_copy(upd[1], acc.at[dst_v.at[1]], ssem[1], add=True)
            pltpu.async_copy(xl.at[src_v.at[0]], rows[0], gsem[0])
            pltpu.async_copy(xl.at[src_v.at[1]], rows[1], gsem[1])

            def do_block(b, par):
                pltpu.make_async_copy(xl.at[src_v.at[b]], rows[par],
                                      gsem[par]).wait()
                pltpu.make_async_copy(upd[par], acc.at[dst_v.at[b]],
                                      ssem[par]).wait()
                lax.fori_loop(0, BB,
                              functools.partial(assemble, off=b * BB, p_r=p_r,
                                                rows_v=rows[par],
                                                upd_v=upd[par]),
                              0)
                pltpu.async_copy(upd[par], acc.at[dst_v.at[b]], ssem[par],
                                 add=True)
                nxt = jnp.minimum(b + 2, NB - 1)
                pltpu.async_copy(xl.at[src_v.at[nxt]], rows[par], gsem[par])

            def pair_body(bp, _):
                do_block(2 * bp, 0)
                do_block(2 * bp + 1, 1)
                return 0

            lax.fori_loop(0, NB // 2, pair_body, 0)
            pltpu.make_async_copy(xl.at[src_v.at[0]], rows[0], gsem[0]).wait()
            pltpu.make_async_copy(xl.at[src_v.at[1]], rows[1], gsem[1]).wait()
            pltpu.make_async_copy(upd[0], acc.at[dst_v.at[0]], ssem[0]).wait()
            pltpu.make_async_copy(upd[1], acc.at[dst_v.at[1]], ssem[1]).wait()
            return 0

        lax.fori_loop(0, R, round_body, 0)
        plsc.subcore_barrier()
        pltpu.sync_copy(acc.at[pl.ds(s * ZR, ZR)],
                        parts.at[c, pl.ds(s * ZR, ZR)])

    return pl.kernel(
        body,
        out_type=jax.ShapeDtypeStruct((2, NACC, UROW), jnp.float32),
        mesh=_MESH,
        compiler_params=_SC_PARAMS,
        scratch_types=[
            pltpu.VMEM_SHARED((NACC, UROW), jnp.float32),
            pltpu.VMEM((NB, BB), jnp.int32),
            pltpu.VMEM((NB, BB), jnp.int32),
            pltpu.VMEM((H, ECR), jnp.float32),
            pltpu.VMEM((BB, CROW), jnp.float32),
            pltpu.VMEM((BB, CROW), jnp.float32),
            pltpu.VMEM((BB, UROW), jnp.float32),
            pltpu.VMEM((BB, UROW), jnp.float32),
            pltpu.SemaphoreType.DMA,
            pltpu.SemaphoreType.DMA,
            pltpu.SemaphoreType.DMA,
            pltpu.SemaphoreType.DMA,
        ],
    )


_p1 = _make_p_kernel(H1, NPAD)
_p2 = _make_p_kernel(1, N)
_s1 = _make_s_kernel(H1, 128, 144)
_s2 = _make_s_kernel(1, 48, 48)


def _mid_body(p0_ref, p1_ref, b1_ref, sexp_ref, w2_ref, as2_ref, ad2_ref,
              xl2_ref, s2_ref, d2_ref, gm2_ref):
    i = pl.program_id(0)
    raw = p0_ref[:, pl.ds(0, 128)] + p1_ref[:, pl.ds(0, 128)]
    den8 = p0_ref[:, pl.ds(128, 16)] + p1_ref[:, pl.ds(128, 16)]
    den128 = jnp.dot(den8, sexp_ref[...], preferred_element_type=jnp.float32)
    h = raw / (den128 + 1e-16) + b1_ref[...]
    h = jnp.where(h > 0, h, jnp.exp(h) - 1.0)
    xl2 = jnp.dot(h, w2_ref[...], preferred_element_type=jnp.float32)
    xl2_ref[...] = xl2
    s2 = jnp.sum(xl2 * as2_ref[...], axis=1, keepdims=True)
    d2 = jnp.sum(xl2 * ad2_ref[...], axis=1, keepdims=True)
    s2_ref[...] = s2
    d2_ref[...] = d2
    cur = jnp.full((8, 128), jnp.max(s2), jnp.float32)

    @pl.when(i == 0)
    def _():
        gm2_ref[...] = cur

    @pl.when(i > 0)
    def _():
        gm2_ref[...] = jnp.maximum(gm2_ref[...], cur)


_mid = pl.pallas_call(
    _mid_body,
    grid=(N // BN,),
    in_specs=[
        pl.BlockSpec((BN, 144), lambda i: (i, 0)),
        pl.BlockSpec((BN, 144), lambda i: (i, 0)),
        pl.BlockSpec((1, 128), lambda i: (0, 0)),
        pl.BlockSpec((16, 128), lambda i: (0, 0)),
        pl.BlockSpec((NFEAT, 48), lambda i: (0, 0)),
        pl.BlockSpec((1, 48), lambda i: (0, 0)),
        pl.BlockSpec((1, 48), lambda i: (0, 0)),
    ],
    out_specs=[
        pl.BlockSpec((BN, 48), lambda i: (i, 0)),
        pl.BlockSpec((BN, 1), lambda i: (i, 0)),
        pl.BlockSpec((BN, 1), lambda i: (i, 0)),
        pl.BlockSpec((8, 128), lambda i: (0, 0)),
    ],
    out_shape=[
        jax.ShapeDtypeStruct((N, 48), jnp.float32),
        jax.ShapeDtypeStruct((N, 1), jnp.float32),
        jax.ShapeDtypeStruct((N, 1), jnp.float32),
        jax.ShapeDtypeStruct((8, 128), jnp.float32),
    ],
)


def _fin_body(q0_ref, q1_ref, b2_ref, out_ref):
    raw = q0_ref[...] + q1_ref[...]
    den = raw[:, 40:41]
    out_ref[...] = raw / (den + 1e-16) + b2_ref[...]


_fin = pl.pallas_call(
    _fin_body,
    grid=(N // BN,),
    in_specs=[
        pl.BlockSpec((BN, 48), lambda i: (i, 0)),
        pl.BlockSpec((BN, 48), lambda i: (i, 0)),
        pl.BlockSpec((1, 48), lambda i: (0, 0)),
    ],
    out_specs=pl.BlockSpec((BN, 48), lambda i: (i, 0)),
    out_shape=jax.ShapeDtypeStruct((N, 48), jnp.float32),
)


def kernel(x, edge_index, W1, att_src1, att_dst1, b1, W2, att_src2, att_dst2,
           b2):
    f32 = jnp.float32
    loop = jnp.arange(N, dtype=edge_index.dtype)
    pad = jnp.zeros((EP - E_REAL,), edge_index.dtype)
    srcp = jnp.concatenate([edge_index[0], loop, pad]).reshape(EP // BB, BB)
    dstp = jnp.concatenate([edge_index[1], loop, pad]).reshape(EP // BB, BB)

    rows = jnp.arange(NFEAT)
    As1 = jnp.zeros((NFEAT, H1), f32).at[rows, rows // HID].set(
        att_src1.reshape(-1))
    Ad1 = jnp.zeros((NFEAT, H1), f32).at[rows, rows // HID].set(
        att_dst1.reshape(-1))
    Sexp = jnp.zeros((16, 128), f32).at[rows // HID, rows].set(1.0)

    xpad = jnp.concatenate([x, jnp.zeros((NPAD - N, NFEAT), f32)])
    xl1, asT1, adT1, gm1 = _pre1(xpad, W1, As1, Ad1)
    gmT1 = gm1[:, :HID]

    ph1 = _p1(srcp, dstp, asT1, adT1, gmT1)
    parts1 = _s1(srcp, dstp, xl1, ph1, jnp.zeros((ZR, 144), f32))

    W2p = jnp.zeros((NFEAT, 48), f32).at[:, :NCLASS].set(W2)
    as2row = jnp.zeros((1, 48), f32).at[:, :NCLASS].set(att_src2)
    ad2row = jnp.zeros((1, 48), f32).at[:, :NCLASS].set(att_dst2)
    xl2, s2, d2, gm2 = _mid(parts1[0, :N], parts1[1, :N], b1[None, :], Sexp,
                            W2p, as2row, ad2row)
    gmT2 = gm2[:1, :16]

    ph2 = _p2(srcp, dstp, s2.reshape(1, N), d2.reshape(1, N), gmT2)
    parts2 = _s2(srcp, dstp, xl2, ph2, jnp.zeros((ZR, 48), f32))

    b2p = jnp.zeros((1, 48), f32).at[:, :NCLASS].set(b2)
    out48 = _fin(parts2[0, :N], parts2[1, :N], b2p)
    return out48[:, :NCLASS]


# speedup vs baseline: 77.5283x; 1.6159x over previous
import functools

import jax
import jax.numpy as jnp
from jax import lax
from jax.experimental import pallas as pl
from jax.experimental.pallas import tpu as pltpu
from jax.experimental.pallas import tpu_sc as plsc

N = 10000
NFEAT = 128
HID = 16
H1 = 8
NCLASS = 40

E_RAW = 320000
E_REAL = E_RAW + N
EP = 331776
NW = 32
R = 6
ECR = EP // (NW * R)
BB = 32
NB = ECR // BB
NACC = 10240
ZR = NACC // 16

BN = 1000
NPAD = 10240
BN1 = 1024


def _lrelu(v):
    return jnp.where(v > 0, v, 0.2 * v)


def _pre1_body(x_ref, w_ref, as_ref, ad_ref, xl_ref, asT_ref, adT_ref, gm_ref):
    i = pl.program_id(0)
    xl = jnp.dot(x_ref[...], w_ref[...], preferred_element_type=jnp.float32)
    xl_ref[...] = xl
    a_s = jnp.dot(xl, as_ref[...], preferred_element_type=jnp.float32)
    a_d = jnp.dot(xl, ad_ref[...], preferred_element_type=jnp.float32)
    asT_ref[...] = a_s.T
    adT_ref[...] = a_d.T
    bm = jnp.max(a_s, axis=0)
    cur = jnp.broadcast_to(bm[:, None], (H1, 128))

    @pl.when(i == 0)
    def _():
        gm_ref[...] = cur

    @pl.when(i > 0)
    def _():
        gm_ref[...] = jnp.maximum(gm_ref[...], cur)


_pre1 = pl.pallas_call(
    _pre1_body,
    grid=(NPAD // BN1,),
    in_specs=[
        pl.BlockSpec((BN1, NFEAT), lambda i: (i, 0)),
        pl.BlockSpec((NFEAT, NFEAT), lambda i: (0, 0)),
        pl.BlockSpec((NFEAT, H1), lambda i: (0, 0)),
        pl.BlockSpec((NFEAT, H1), lambda i: (0, 0)),
    ],
    out_specs=[
        pl.BlockSpec((BN1, NFEAT), lambda i: (i, 0)),
        pl.BlockSpec((H1, BN1), lambda i: (0, i)),
        pl.BlockSpec((H1, BN1), lambda i: (0, i)),
        pl.BlockSpec((H1, 128), lambda i: (0, 0)),
    ],
    out_shape=[
        jax.ShapeDtypeStruct((NPAD, NFEAT), jnp.float32),
        jax.ShapeDtypeStruct((H1, NPAD), jnp.float32),
        jax.ShapeDtypeStruct((H1, NPAD), jnp.float32),
        jax.ShapeDtypeStruct((H1, 128), jnp.float32),
    ],
)


WEC = EP // NW
RNB = WEC // BB

_MESH = plsc.VectorSubcoreMesh(core_axis_name="c", subcore_axis_name="s",
                               num_cores=2, num_subcores=16)
_SC_PARAMS = pltpu.CompilerParams(use_tc_tiling_on_sc=False,
                                  needs_layout_passes=False)


def _make_p_kernel(H, TN):
    def body(srcp, dstp, asT, adT, gmaxT, ph,
             src_v, dst_v, as_v, ad_v, gm_v, p_r):
        c = lax.axis_index("c")
        s = lax.axis_index("s")
        w = s * 2 + c
        pltpu.sync_copy(gmaxT, gm_v)
        pltpu.sync_copy(srcp.at[pl.ds(w * RNB, RNB)], src_v)
        pltpu.sync_copy(dstp.at[pl.ds(w * RNB, RNB)], dst_v)
        base = w * WEC
        for h in range(H):
            pltpu.sync_copy(asT.at[h], as_v)
            pltpu.sync_copy(adT.at[h], ad_v)
            gmv = gm_v[h]

            @plsc.parallel_loop(0, RNB, unroll=2)
            def p_body(b, gmv=gmv):
                for j in range(BB // 16):
                    si = src_v[b, pl.ds(j * 16, 16)]
                    di = dst_v[b, pl.ds(j * 16, 16)]
                    sv = plsc.load_gather(as_v, [si])
                    dv = plsc.load_gather(ad_v, [di])
                    al = _lrelu(sv + dv)
                    uu = _lrelu(gmv + dv)
                    p = jnp.exp(al - uu)
                    eid = base + b * BB + j * 16 + lax.iota(jnp.int32, 16)
                    p = jnp.where(eid < E_REAL, p,
                                  jnp.zeros((16,), jnp.float32))
                    p_r[pl.ds(b * BB + j * 16, 16)] = p
            pltpu.sync_copy(p_r, ph.at[h, pl.ds(base, WEC)])

    return pl.kernel(
        body,
        out_type=jax.ShapeDtypeStruct((H, EP), jnp.float32),
        mesh=_MESH,
        compiler_params=_SC_PARAMS,
        scratch_types=[
            pltpu.VMEM((RNB, BB), jnp.int32),
            pltpu.VMEM((RNB, BB), jnp.int32),
            pltpu.VMEM((TN,), jnp.float32),
            pltpu.VMEM((TN,), jnp.float32),
            pltpu.VMEM((H, 16), jnp.float32),
            pltpu.VMEM((WEC,), jnp.float32),
        ],
    )


def _make_s_kernel(H, CROW, UROW, BBL):
    NBL = ECR // BBL

    def assemble(e, off, p_r, rows_v, upd_v):
        pcol = jnp.full((16,), off + e, jnp.int32)
        if H == H1:
            for h in range(H):
                pb = plsc.load_gather(p_r, [jnp.full((16,), h, jnp.int32), pcol])
                upd_v[e, pl.ds(h * 16, 16)] = rows_v[e, pl.ds(h * 16, 16)] * pb
            hidx = lax.iota(jnp.int32, 16)
            pall = plsc.load_gather(p_r, [jnp.minimum(hidx, H - 1), pcol])
            upd_v[e, pl.ds(H * 16, 16)] = jnp.where(
                hidx < H, pall, jnp.zeros((16,), jnp.float32))
        else:
            pb = plsc.load_gather(p_r, [jnp.zeros((16,), jnp.int32), pcol])
            onehot = jnp.where(lax.iota(jnp.int32, 16) == 8,
                               jnp.ones((16,), jnp.float32),
                               jnp.zeros((16,), jnp.float32))
            upd_v[e, pl.ds(0, 16)] = rows_v[e, pl.ds(0, 16)] * pb
            upd_v[e, pl.ds(16, 16)] = rows_v[e, pl.ds(16, 16)] * pb
            upd_v[e, pl.ds(32, 16)] = (rows_v[e, pl.ds(32, 16)] + onehot) * pb

    def body(srcp, dstp, xl, ph, zrow, parts,
             acc, src_v, dst_v, p_r, rows0, rows1, upd0, upd1,
             gsem0, gsem1, ssem0, ssem1):
        c = lax.axis_index("c")
        s = lax.axis_index("s")
        w = s * 2 + c
        rows = (rows0, rows1)
        upd = (upd0, upd1)
        gsem = (gsem0, gsem1)
        ssem = (ssem0, ssem1)
        pltpu.sync_copy(zrow, acc.at[pl.ds(s * ZR, ZR)])
        plsc.subcore_barrier()

        def round_body(r, _):
            rowbase = (w * R + r) * NBL
            base = rowbase * BBL
            pltpu.sync_copy(srcp.at[pl.ds(rowbase, NBL)], src_v)
            pltpu.sync_copy(dstp.at[pl.ds(rowbase, NBL)], dst_v)
            pltpu.sync_copy(ph.at[:, pl.ds(base, ECR)], p_r)

            @plsc.parallel_loop(0, BBL)
            def zero_upd(e):
                zv = jnp.zeros((16,), jnp.float32)
                for k in range(UROW // 16):
                    upd[0][e, pl.ds(k * 16, 16)] = zv
                    upd[1][e, pl.ds(k * 16, 16)] = zv
            pltpu.async_copy(upd[0], acc.at[dst_v.at[0]], ssem[0], add=True)
            pltpu.async_copy(upd[1], acc.at[dst_v.at[1]], ssem[1], add=True)
            pltpu.async_copy(xl.at[src_v.at[0]], rows[0], gsem[0])
            pltpu.async_copy(xl.at[src_v.at[1]], rows[1], gsem[1])

            def do_block(b, par):
                pltpu.make_async_copy(xl.at[src_v.at[b]], rows[par],
                                      gsem[par]).wait()
                pltpu.make_async_copy(upd[par], acc.at[dst_v.at[b]],
                                      ssem[par]).wait()
                plsc.parallel_loop(0, BBL)(
                    functools.partial(assemble, off=b * BBL, p_r=p_r,
                                      rows_v=rows[par], upd_v=upd[par]))
                pltpu.async_copy(upd[par], acc.at[dst_v.at[b]], ssem[par],
                                 add=True)
                nxt = jnp.minimum(b + 2, NBL - 1)
                pltpu.async_copy(xl.at[src_v.at[nxt]], rows[par], gsem[par])

            def pair_body(bp, _):
                do_block(2 * bp, 0)
                do_block(2 * bp + 1, 1)
                return 0

            lax.fori_loop(0, NBL // 2, pair_body, 0)
            pltpu.make_async_copy(xl.at[src_v.at[0]], rows[0], gsem[0]).wait()
            pltpu.make_async_copy(xl.at[src_v.at[1]], rows[1], gsem[1]).wait()
            pltpu.make_async_copy(upd[0], acc.at[dst_v.at[0]], ssem[0]).wait()
            pltpu.make_async_copy(upd[1], acc.at[dst_v.at[1]], ssem[1]).wait()
            return 0

        lax.fori_loop(0, R, round_body, 0)
        plsc.subcore_barrier()
        pltpu.sync_copy(acc.at[pl.ds(s * ZR, ZR)],
                        parts.at[c, pl.ds(s * ZR, ZR)])

    return pl.kernel(
        body,
        out_type=jax.ShapeDtypeStruct((2, NACC, UROW), jnp.float32),
        mesh=_MESH,
        compiler_params=_SC_PARAMS,
        scratch_types=[
            pltpu.VMEM_SHARED((NACC, UROW), jnp.float32),
            pltpu.VMEM((NBL, BBL), jnp.int32),
            pltpu.VMEM((NBL, BBL), jnp.int32),
            pltpu.VMEM((H, ECR), jnp.float32),
            pltpu.VMEM((BBL, CROW), jnp.float32),
            pltpu.VMEM((BBL, CROW), jnp.float32),
            pltpu.VMEM((BBL, UROW), jnp.float32),
            pltpu.VMEM((BBL, UROW), jnp.float32),
            pltpu.SemaphoreType.DMA,
            pltpu.SemaphoreType.DMA,
            pltpu.SemaphoreType.DMA,
            pltpu.SemaphoreType.DMA,
        ],
    )


_p1 = _make_p_kernel(H1, NPAD)
_p2 = _make_p_kernel(1, N)
_s1 = _make_s_kernel(H1, 128, 144, 32)
_s2 = _make_s_kernel(1, 48, 48, 96)


def _mid_body(p0_ref, p1_ref, b1_ref, sexp_ref, w2_ref, as2_ref, ad2_ref,
              xl2_ref, s2_ref, d2_ref, gm2_ref):
    i = pl.program_id(0)
    raw = p0_ref[:, pl.ds(0, 128)] + p1_ref[:, pl.ds(0, 128)]
    den8 = p0_ref[:, pl.ds(128, 16)] + p1_ref[:, pl.ds(128, 16)]
    den128 = jnp.dot(den8, sexp_ref[...], preferred_element_type=jnp.float32)
    h = raw / (den128 + 1e-16) + b1_ref[...]
    h = jnp.where(h > 0, h, jnp.exp(h) - 1.0)
    xl2 = jnp.dot(h, w2_ref[...], preferred_element_type=jnp.float32)
    xl2_ref[...] = xl2
    s2 = jnp.sum(xl2 * as2_ref[...], axis=1, keepdims=True)
    d2 = jnp.sum(xl2 * ad2_ref[...], axis=1, keepdims=True)
    s2_ref[...] = s2
    d2_ref[...] = d2
    cur = jnp.full((8, 128), jnp.max(s2), jnp.float32)

    @pl.when(i == 0)
    def _():
        gm2_ref[...] = cur

    @pl.when(i > 0)
    def _():
        gm2_ref[...] = jnp.maximum(gm2_ref[...], cur)


_mid = pl.pallas_call(
    _mid_body,
    grid=(N // BN,),
    in_specs=[
        pl.BlockSpec((BN, 144), lambda i: (i, 0)),
        pl.BlockSpec((BN, 144), lambda i: (i, 0)),
        pl.BlockSpec((1, 128), lambda i: (0, 0)),
        pl.BlockSpec((16, 128), lambda i: (0, 0)),
        pl.BlockSpec((NFEAT, 48), lambda i: (0, 0)),
        pl.BlockSpec((1, 48), lambda i: (0, 0)),
        pl.BlockSpec((1, 48), lambda i: (0, 0)),
    ],
    out_specs=[
        pl.BlockSpec((BN, 48), lambda i: (i, 0)),
        pl.BlockSpec((BN, 1), lambda i: (i, 0)),
        pl.BlockSpec((BN, 1), lambda i: (i, 0)),
        pl.BlockSpec((8, 128), lambda i: (0, 0)),
    ],
    out_shape=[
        jax.ShapeDtypeStruct((N, 48), jnp.float32),
        jax.ShapeDtypeStruct((N, 1), jnp.float32),
        jax.ShapeDtypeStruct((N, 1), jnp.float32),
        jax.ShapeDtypeStruct((8, 128), jnp.float32),
    ],
)


def _fin_body(q0_ref, q1_ref, b2_ref, out_ref):
    raw = q0_ref[...] + q1_ref[...]
    den = raw[:, 40:41]
    out_ref[...] = raw / (den + 1e-16) + b2_ref[...]


_fin = pl.pallas_call(
    _fin_body,
    grid=(N // BN,),
    in_specs=[
        pl.BlockSpec((BN, 48), lambda i: (i, 0)),
        pl.BlockSpec((BN, 48), lambda i: (i, 0)),
        pl.BlockSpec((1, 48), lambda i: (0, 0)),
    ],
    out_specs=pl.BlockSpec((BN, 48), lambda i: (i, 0)),
    out_shape=jax.ShapeDtypeStruct((N, 48), jnp.float32),
)


def kernel(x, edge_index, W1, att_src1, att_dst1, b1, W2, att_src2, att_dst2,
           b2):
    f32 = jnp.float32
    loop = jnp.arange(N, dtype=edge_index.dtype)
    pad = jnp.zeros((EP - E_REAL,), edge_index.dtype)
    srcf = jnp.concatenate([edge_index[0], loop, pad])
    dstf = jnp.concatenate([edge_index[1], loop, pad])
    srcp = srcf.reshape(EP // BB, BB)
    dstp = dstf.reshape(EP // BB, BB)
    srcp96 = srcf.reshape(EP // 96, 96)
    dstp96 = dstf.reshape(EP // 96, 96)

    rows = jnp.arange(NFEAT)
    As1 = jnp.zeros((NFEAT, H1), f32).at[rows, rows // HID].set(
        att_src1.reshape(-1))
    Ad1 = jnp.zeros((NFEAT, H1), f32).at[rows, rows // HID].set(
        att_dst1.reshape(-1))
    Sexp = jnp.zeros((16, 128), f32).at[rows // HID, rows].set(1.0)

    xpad = jnp.concatenate([x, jnp.zeros((NPAD - N, NFEAT), f32)])
    xl1, asT1, adT1, gm1 = _pre1(xpad, W1, As1, Ad1)
    gmT1 = gm1[:, :HID]

    ph1 = _p1(srcp, dstp, asT1, adT1, gmT1)
    parts1 = _s1(srcp, dstp, xl1, ph1, jnp.zeros((ZR, 144), f32))

    W2p = jnp.zeros((NFEAT, 48), f32).at[:, :NCLASS].set(W2)
    as2row = jnp.zeros((1, 48), f32).at[:, :NCLASS].set(att_src2)
    ad2row = jnp.zeros((1, 48), f32).at[:, :NCLASS].set(att_dst2)
    xl2, s2, d2, gm2 = _mid(parts1[0, :N], parts1[1, :N], b1[None, :], Sexp,
                            W2p, as2row, ad2row)
    gmT2 = gm2[:1, :16]

    ph2 = _p2(srcp, dstp, s2.reshape(1, N), d2.reshape(1, N), gmT2)
    parts2 = _s2(srcp96, dstp96, xl2, ph2, jnp.zeros((ZR, 48), f32))

    b2p = jnp.zeros((1, 48), f32).at[:, :NCLASS].set(b2)
    out48 = _fin(parts2[0, :N], parts2[1, :N], b2p)
    return out48[:, :NCLASS]


# speedup vs baseline: 77.8028x; 1.0035x over previous
import functools

import jax
import jax.numpy as jnp
from jax import lax
from jax.experimental import pallas as pl
from jax.experimental.pallas import tpu as pltpu
from jax.experimental.pallas import tpu_sc as plsc

N = 10000
NFEAT = 128
HID = 16
H1 = 8
NCLASS = 40

E_RAW = 320000
E_REAL = E_RAW + N
EP = 331776
NW = 32
R = 6
ECR = EP // (NW * R)
BB = 32
NB = ECR // BB
NACC = 10240
ZR = NACC // 16

BN = 1000
NPAD = 10240
BN1 = 1024


def _lrelu(v):
    return jnp.where(v > 0, v, 0.2 * v)


def _pre1_body(x_ref, w_ref, as_ref, ad_ref, xl_ref, asT_ref, adT_ref, gm_ref):
    i = pl.program_id(0)
    xl = jnp.dot(x_ref[...], w_ref[...], preferred_element_type=jnp.float32)
    xl_ref[...] = xl
    a_s = jnp.dot(xl, as_ref[...], preferred_element_type=jnp.float32)
    a_d = jnp.dot(xl, ad_ref[...], preferred_element_type=jnp.float32)
    asT_ref[...] = a_s.T
    adT_ref[...] = a_d.T
    bm = jnp.max(a_s, axis=0)
    cur = jnp.broadcast_to(bm[:, None], (H1, 128))

    @pl.when(i == 0)
    def _():
        gm_ref[...] = cur

    @pl.when(i > 0)
    def _():
        gm_ref[...] = jnp.maximum(gm_ref[...], cur)


_pre1 = pl.pallas_call(
    _pre1_body,
    grid=(NPAD // BN1,),
    in_specs=[
        pl.BlockSpec((BN1, NFEAT), lambda i: (i, 0)),
        pl.BlockSpec((NFEAT, NFEAT), lambda i: (0, 0)),
        pl.BlockSpec((NFEAT, H1), lambda i: (0, 0)),
        pl.BlockSpec((NFEAT, H1), lambda i: (0, 0)),
    ],
    out_specs=[
        pl.BlockSpec((BN1, NFEAT), lambda i: (i, 0)),
        pl.BlockSpec((H1, BN1), lambda i: (0, i)),
        pl.BlockSpec((H1, BN1), lambda i: (0, i)),
        pl.BlockSpec((H1, 128), lambda i: (0, 0)),
    ],
    out_shape=[
        jax.ShapeDtypeStruct((NPAD, NFEAT), jnp.float32),
        jax.ShapeDtypeStruct((H1, NPAD), jnp.float32),
        jax.ShapeDtypeStruct((H1, NPAD), jnp.float32),
        jax.ShapeDtypeStruct((H1, 128), jnp.float32),
    ],
)


WEC = EP // NW
RNB = WEC // BB

_MESH = plsc.VectorSubcoreMesh(core_axis_name="c", subcore_axis_name="s",
                               num_cores=2, num_subcores=16)
_SC_PARAMS = pltpu.CompilerParams(use_tc_tiling_on_sc=False,
                                  needs_layout_passes=False)


def _make_p_kernel(H, TN):
    def body(srcp, dstp, asT, adT, gmaxT, ph,
             src_v, dst_v, as_v, ad_v, gm_v, p_r):
        c = lax.axis_index("c")
        s = lax.axis_index("s")
        w = s * 2 + c
        pltpu.sync_copy(gmaxT, gm_v)
        pltpu.sync_copy(srcp.at[pl.ds(w * RNB, RNB)], src_v)
        pltpu.sync_copy(dstp.at[pl.ds(w * RNB, RNB)], dst_v)
        base = w * WEC
        for h in range(H):
            pltpu.sync_copy(asT.at[h], as_v)
            pltpu.sync_copy(adT.at[h], ad_v)
            gmv = gm_v[h]

            @plsc.parallel_loop(0, RNB, unroll=2)
            def p_body(b, gmv=gmv):
                for j in range(BB // 16):
                    si = src_v[b, pl.ds(j * 16, 16)]
                    di = dst_v[b, pl.ds(j * 16, 16)]
                    sv = plsc.load_gather(as_v, [si])
                    dv = plsc.load_gather(ad_v, [di])
                    al = _lrelu(sv + dv)
                    uu = _lrelu(gmv + dv)
                    p = jnp.exp(al - uu)
                    eid = base + b * BB + j * 16 + lax.iota(jnp.int32, 16)
                    p = jnp.where(eid < E_REAL, p,
                                  jnp.zeros((16,), jnp.float32))
                    p_r[pl.ds(b * BB + j * 16, 16)] = p
            pltpu.sync_copy(p_r, ph.at[h, pl.ds(base, WEC)])

    return pl.kernel(
        body,
        out_type=jax.ShapeDtypeStruct((H, EP), jnp.float32),
        mesh=_MESH,
        compiler_params=_SC_PARAMS,
        scratch_types=[
            pltpu.VMEM((RNB, BB), jnp.int32),
            pltpu.VMEM((RNB, BB), jnp.int32),
            pltpu.VMEM((TN,), jnp.float32),
            pltpu.VMEM((TN,), jnp.float32),
            pltpu.VMEM((H, 16), jnp.float32),
            pltpu.VMEM((WEC,), jnp.float32),
        ],
    )


def _make_s_kernel(H, CROW, UROW, BBL, inline_p=False):
    NBL = ECR // BBL

    def assemble(e, off, p_r, rows_v, upd_v):
        pcol = jnp.full((16,), off + e, jnp.int32)
        if H == H1:
            for h in range(H):
                pb = plsc.load_gather(p_r, [jnp.full((16,), h, jnp.int32), pcol])
                upd_v[e, pl.ds(h * 16, 16)] = rows_v[e, pl.ds(h * 16, 16)] * pb
            hidx = lax.iota(jnp.int32, 16)
            pall = plsc.load_gather(p_r, [jnp.minimum(hidx, H - 1), pcol])
            upd_v[e, pl.ds(H * 16, 16)] = jnp.where(
                hidx < H, pall, jnp.zeros((16,), jnp.float32))
        else:
            pb = plsc.load_gather(p_r, [jnp.zeros((16,), jnp.int32), pcol])
            onehot = jnp.where(lax.iota(jnp.int32, 16) == 8,
                               jnp.ones((16,), jnp.float32),
                               jnp.zeros((16,), jnp.float32))
            upd_v[e, pl.ds(0, 16)] = rows_v[e, pl.ds(0, 16)] * pb
            upd_v[e, pl.ds(16, 16)] = rows_v[e, pl.ds(16, 16)] * pb
            upd_v[e, pl.ds(32, 16)] = (rows_v[e, pl.ds(32, 16)] + onehot) * pb

    def body(*refs):
        if inline_p:
            (srcp, dstp, xl, asT, adT, gmaxT, zrow, parts,
             acc, src_v, dst_v, p_r, rows0, rows1, upd0, upd1,
             gsem0, gsem1, ssem0, ssem1, as_v, ad_v, gm_v) = refs
            ph = None
        else:
            (srcp, dstp, xl, ph, zrow, parts,
             acc, src_v, dst_v, p_r, rows0, rows1, upd0, upd1,
             gsem0, gsem1, ssem0, ssem1) = refs
        c = lax.axis_index("c")
        s = lax.axis_index("s")
        w = s * 2 + c
        if inline_p:
            pltpu.sync_copy(asT.at[0], as_v)
            pltpu.sync_copy(adT.at[0], ad_v)
            pltpu.sync_copy(gmaxT, gm_v)
        rows = (rows0, rows1)
        upd = (upd0, upd1)
        gsem = (gsem0, gsem1)
        ssem = (ssem0, ssem1)
        pltpu.sync_copy(zrow, acc.at[pl.ds(s * ZR, ZR)])
        plsc.subcore_barrier()

        def round_body(r, _):
            rowbase = (w * R + r) * NBL
            base = rowbase * BBL
            pltpu.sync_copy(srcp.at[pl.ds(rowbase, NBL)], src_v)
            pltpu.sync_copy(dstp.at[pl.ds(rowbase, NBL)], dst_v)
            if not inline_p:
                pltpu.sync_copy(ph.at[:, pl.ds(base, ECR)], p_r)
            else:
                gmv = gm_v[0]

                @plsc.parallel_loop(0, NBL, unroll=2)
                def p_comp(b, gmv=gmv):
                    for j in range(BBL // 16):
                        si = src_v[b, pl.ds(j * 16, 16)]
                        di = dst_v[b, pl.ds(j * 16, 16)]
                        sv = plsc.load_gather(as_v, [si])
                        dv = plsc.load_gather(ad_v, [di])
                        al = _lrelu(sv + dv)
                        uu = _lrelu(gmv + dv)
                        p = jnp.exp(al - uu)
                        eid = (base + b * BBL + j * 16
                               + lax.iota(jnp.int32, 16))
                        p = jnp.where(eid < E_REAL, p,
                                      jnp.zeros((16,), jnp.float32))
                        p_r[0, pl.ds(b * BBL + j * 16, 16)] = p

            @plsc.parallel_loop(0, BBL)
            def zero_upd(e):
                zv = jnp.zeros((16,), jnp.float32)
                for k in range(UROW // 16):
                    upd[0][e, pl.ds(k * 16, 16)] = zv
                    upd[1][e, pl.ds(k * 16, 16)] = zv
            pltpu.async_copy(upd[0], acc.at[dst_v.at[0]], ssem[0], add=True)
            pltpu.async_copy(upd[1], acc.at[dst_v.at[1]], ssem[1], add=True)
            pltpu.async_copy(xl.at[src_v.at[0]], rows[0], gsem[0])
            pltpu.async_copy(xl.at[src_v.at[1]], rows[1], gsem[1])

            def do_block(b, par):
                pltpu.make_async_copy(xl.at[src_v.at[b]], rows[par],
                                      gsem[par]).wait()
                pltpu.make_async_copy(upd[par], acc.at[dst_v.at[b]],
                                      ssem[par]).wait()
                plsc.parallel_loop(0, BBL)(
                    functools.partial(assemble, off=b * BBL, p_r=p_r,
                                      rows_v=rows[par], upd_v=upd[par]))
                pltpu.async_copy(upd[par], acc.at[dst_v.at[b]], ssem[par],
                                 add=True)
                nxt = jnp.minimum(b + 2, NBL - 1)
                pltpu.async_copy(xl.at[src_v.at[nxt]], rows[par], gsem[par])

            def pair_body(bp, _):
                do_block(2 * bp, 0)
                do_block(2 * bp + 1, 1)
                return 0

            lax.fori_loop(0, NBL // 2, pair_body, 0)
            pltpu.make_async_copy(xl.at[src_v.at[0]], rows[0], gsem[0]).wait()
            pltpu.make_async_copy(xl.at[src_v.at[1]], rows[1], gsem[1]).wait()
            pltpu.make_async_copy(upd[0], acc.at[dst_v.at[0]], ssem[0]).wait()
            pltpu.make_async_copy(upd[1], acc.at[dst_v.at[1]], ssem[1]).wait()
            return 0

        lax.fori_loop(0, R, round_body, 0)
        plsc.subcore_barrier()
        pltpu.sync_copy(acc.at[pl.ds(s * ZR, ZR)],
                        parts.at[c, pl.ds(s * ZR, ZR)])

    scratch = [
        pltpu.VMEM_SHARED((NACC, UROW), jnp.float32),
        pltpu.VMEM((NBL, BBL), jnp.int32),
        pltpu.VMEM((NBL, BBL), jnp.int32),
        pltpu.VMEM((H, ECR), jnp.float32),
        pltpu.VMEM((BBL, CROW), jnp.float32),
        pltpu.VMEM((BBL, CROW), jnp.float32),
        pltpu.VMEM((BBL, UROW), jnp.float32),
        pltpu.VMEM((BBL, UROW), jnp.float32),
        pltpu.SemaphoreType.DMA,
        pltpu.SemaphoreType.DMA,
        pltpu.SemaphoreType.DMA,
        pltpu.SemaphoreType.DMA,
    ]
    if inline_p:
        scratch += [
            pltpu.VMEM((N,), jnp.float32),
            pltpu.VMEM((N,), jnp.float32),
            pltpu.VMEM((H, 16), jnp.float32),
        ]
    return pl.kernel(
        body,
        out_type=jax.ShapeDtypeStruct((2, NACC, UROW), jnp.float32),
        mesh=_MESH,
        compiler_params=_SC_PARAMS,
        scratch_types=scratch,
    )


_p1 = _make_p_kernel(H1, NPAD)
_s1 = _make_s_kernel(H1, 128, 144, 32)
_s2 = _make_s_kernel(1, 48, 48, 96, inline_p=True)


def _mid_body(p0_ref, p1_ref, b1_ref, sexp_ref, w2_ref, as2_ref, ad2_ref,
              xl2_ref, s2_ref, d2_ref, gm2_ref):
    i = pl.program_id(0)
    raw = p0_ref[:, pl.ds(0, 128)] + p1_ref[:, pl.ds(0, 128)]
    den8 = p0_ref[:, pl.ds(128, 16)] + p1_ref[:, pl.ds(128, 16)]
    den128 = jnp.dot(den8, sexp_ref[...], preferred_element_type=jnp.float32)
    h = raw / (den128 + 1e-16) + b1_ref[...]
    h = jnp.where(h > 0, h, jnp.exp(h) - 1.0)
    xl2 = jnp.dot(h, w2_ref[...], preferred_element_type=jnp.float32)
    xl2_ref[...] = xl2
    s2 = jnp.sum(xl2 * as2_ref[...], axis=1, keepdims=True)
    d2 = jnp.sum(xl2 * ad2_ref[...], axis=1, keepdims=True)
    s2_ref[...] = s2
    d2_ref[...] = d2
    cur = jnp.full((8, 128), jnp.max(s2), jnp.float32)

    @pl.when(i == 0)
    def _():
        gm2_ref[...] = cur

    @pl.when(i > 0)
    def _():
        gm2_ref[...] = jnp.maximum(gm2_ref[...], cur)


_mid = pl.pallas_call(
    _mid_body,
    grid=(N // BN,),
    in_specs=[
        pl.BlockSpec((BN, 144), lambda i: (i, 0)),
        pl.BlockSpec((BN, 144), lambda i: (i, 0)),
        pl.BlockSpec((1, 128), lambda i: (0, 0)),
        pl.BlockSpec((16, 128), lambda i: (0, 0)),
        pl.BlockSpec((NFEAT, 48), lambda i: (0, 0)),
        pl.BlockSpec((1, 48), lambda i: (0, 0)),
        pl.BlockSpec((1, 48), lambda i: (0, 0)),
    ],
    out_specs=[
        pl.BlockSpec((BN, 48), lambda i: (i, 0)),
        pl.BlockSpec((BN, 1), lambda i: (i, 0)),
        pl.BlockSpec((BN, 1), lambda i: (i, 0)),
        pl.BlockSpec((8, 128), lambda i: (0, 0)),
    ],
    out_shape=[
        jax.ShapeDtypeStruct((N, 48), jnp.float32),
        jax.ShapeDtypeStruct((N, 1), jnp.float32),
        jax.ShapeDtypeStruct((N, 1), jnp.float32),
        jax.ShapeDtypeStruct((8, 128), jnp.float32),
    ],
)


def _fin_body(q0_ref, q1_ref, b2_ref, out_ref):
    raw = q0_ref[...] + q1_ref[...]
    den = raw[:, 40:41]
    out_ref[...] = raw / (den + 1e-16) + b2_ref[...]


_fin = pl.pallas_call(
    _fin_body,
    grid=(N // BN,),
    in_specs=[
        pl.BlockSpec((BN, 48), lambda i: (i, 0)),
        pl.BlockSpec((BN, 48), lambda i: (i, 0)),
        pl.BlockSpec((1, 48), lambda i: (0, 0)),
    ],
    out_specs=pl.BlockSpec((BN, 48), lambda i: (i, 0)),
    out_shape=jax.ShapeDtypeStruct((N, 48), jnp.float32),
)


def kernel(x, edge_index, W1, att_src1, att_dst1, b1, W2, att_src2, att_dst2,
           b2):
    f32 = jnp.float32
    loop = jnp.arange(N, dtype=edge_index.dtype)
    pad = jnp.zeros((EP - E_REAL,), edge_index.dtype)
    srcf = jnp.concatenate([edge_index[0], loop, pad])
    dstf = jnp.concatenate([edge_index[1], loop, pad])
    srcp = srcf.reshape(EP // BB, BB)
    dstp = dstf.reshape(EP // BB, BB)
    srcp96 = srcf.reshape(EP // 96, 96)
    dstp96 = dstf.reshape(EP // 96, 96)

    rows = jnp.arange(NFEAT)
    As1 = jnp.zeros((NFEAT, H1), f32).at[rows, rows // HID].set(
        att_src1.reshape(-1))
    Ad1 = jnp.zeros((NFEAT, H1), f32).at[rows, rows // HID].set(
        att_dst1.reshape(-1))
    Sexp = jnp.zeros((16, 128), f32).at[rows // HID, rows].set(1.0)

    xpad = jnp.concatenate([x, jnp.zeros((NPAD - N, NFEAT), f32)])
    xl1, asT1, adT1, gm1 = _pre1(xpad, W1, As1, Ad1)
    gmT1 = gm1[:, :HID]

    ph1 = _p1(srcp, dstp, asT1, adT1, gmT1)
    parts1 = _s1(srcp, dstp, xl1, ph1, jnp.zeros((ZR, 144), f32))

    W2p = jnp.zeros((NFEAT, 48), f32).at[:, :NCLASS].set(W2)
    as2row = jnp.zeros((1, 48), f32).at[:, :NCLASS].set(att_src2)
    ad2row = jnp.zeros((1, 48), f32).at[:, :NCLASS].set(att_dst2)
    xl2, s2, d2, gm2 = _mid(parts1[0, :N], parts1[1, :N], b1[None, :], Sexp,
                            W2p, as2row, ad2row)
    gmT2 = gm2[:1, :16]

    parts2 = _s2(srcp96, dstp96, xl2, s2.reshape(1, N), d2.reshape(1, N),
                 gmT2, jnp.zeros((ZR, 48), f32))

    b2p = jnp.zeros((1, 48), f32).at[:, :NCLASS].set(b2)
    out48 = _fin(parts2[0, :N], parts2[1, :N], b2p)
    return out48[:, :NCLASS]
